# baseline (speedup 1.0000x reference)
import functools

import jax
import jax.numpy as jnp
from jax import lax
from jax.experimental import pallas as pl
from jax.experimental.pallas import tpu as pltpu
from jax.experimental.pallas import tpu_sc as plsc

B, S, D = 1024, 200, 128
PAD = 21
VOCAB, MAXPOS = 32, 256
EPS = 1e-12
NC, NS = 2, 16
NW = NC * NS
RW = B // NW
SPAD = 208
NCHUNK = SPAD // 16
NK = D // 16


def _perm(x, idx):
    return x.at[idx].get(mode="promise_in_bounds")


def _splat_sum(x):
    lane = lax.iota(jnp.int32, 16)
    for k in range(4):
        x = x + _perm(x, lane ^ (1 << k))
    return x


def _prefix_sum(x):
    lane = lax.iota(jnp.int32, 16)
    for k in range(4):
        sh = _perm(x, jnp.maximum(lane - (1 << k), 0))
        x = x + jnp.where(lane >= (1 << k), sh, jnp.int32(0))
    return x


def _rsqrt(x):
    i = lax.bitcast_convert_type(x, jnp.int32)
    i = jnp.int32(0x5F3759DF) - lax.shift_right_logical(i, 1)
    y = lax.bitcast_convert_type(i, jnp.float32)
    for _ in range(3):
        y = y * (1.5 - 0.5 * x * y * y)
    return y


def _body(src_h, aa_h, pos_h, ga_h, be_h, mr_h, out_h,
          aa_v, pos_v, ga_v, be_v, mr_v, src_v, poff_v, obuf,
          sem_in, sem_out):
    wid = lax.axis_index("s") * NC + lax.axis_index("c")
    base_row = wid * RW

    stage = [
        pltpu.make_async_copy(aa_h, aa_v, sem_in),
        pltpu.make_async_copy(pos_h, pos_v, sem_in),
        pltpu.make_async_copy(ga_h, ga_v, sem_in),
        pltpu.make_async_copy(be_h, be_v, sem_in),
        pltpu.make_async_copy(mr_h, mr_v.at[pl.ds(0, 2 * VOCAB * MAXPOS)],
                              sem_in),
        pltpu.make_async_copy(src_h.at[pl.ds(base_row * S, RW * S)],
                              src_v.at[pl.ds(0, RW * S)], sem_in),
    ]
    for c in stage:
        c.start()
    for c in stage:
        c.wait()

    gvec = [ga_v[pl.ds(k * 16, 16)] for k in range(NK)]
    bvec = [be_v[pl.ds(k * 16, 16)] for k in range(NK)]
    last = jnp.full((16,), 15, jnp.int32)

    def row_body(r, _):
        bidx = lax.rem(r, 2)

        @pl.when(r >= 2)
        def _wait_prev():
            pltpu.make_async_copy(obuf.at[bidx, pl.ds(0, S)],
                                  out_h.at[base_row + r - 2], sem_out).wait()

        carry = jnp.zeros((16,), jnp.int32)
        for i in range(NCHUNK):
            sv = src_v[pl.ds(r * S + i * 16, 16)]
            m = jnp.where(sv == PAD, jnp.int32(0), jnp.int32(1))
            c = _prefix_sum(m) + carry
            poff_v[r, pl.ds(i * 16, 16)] = jnp.where(m == 1, c, jnp.int32(0))
            carry = _perm(c, last)

        def chunk_body(i, _2):
            tb = i * 16
            sv = jnp.minimum(jnp.maximum(src_v[pl.ds(r * S + tb, 16)],
                                         jnp.int32(0)), jnp.int32(VOCAB - 1))
            pv = jnp.minimum(jnp.maximum(poff_v[r, pl.ds(tb, 16)],
                                         jnp.int32(0)), jnp.int32(MAXPOS - 1))
            iv = sv * jnp.int32(2 * MAXPOS) + pv * 2
            zero16 = jnp.zeros((16,), jnp.int32)
            one16 = jnp.full((16,), 1, jnp.int32)
            for l in range(16):
                s = sv[l]
                p = pv[l]
                mr = mr_v[pl.ds(iv[l], 16)]
                ml = _perm(mr, zero16)
                rl = _perm(mr, one16)
                for k in range(NK):
                    e = aa_v[s, pl.ds(k * 16, 16)] + pos_v[p, pl.ds(k * 16, 16)]
                    obuf[bidx, tb + l, pl.ds(k * 16, 16)] = \
                        (e - ml) * rl * gvec[k] + bvec[k]
            return 0

        lax.fori_loop(0, NCHUNK, chunk_body, 0)
        pltpu.make_async_copy(obuf.at[bidx, pl.ds(0, S)],
                              out_h.at[base_row + r], sem_out).start()
        return 0

    lax.fori_loop(0, RW, row_body, 0)
    pltpu.make_async_copy(obuf.at[0, pl.ds(0, S)],
                          out_h.at[base_row + RW - 2], sem_out).wait()
    pltpu.make_async_copy(obuf.at[1, pl.ds(0, S)],
                          out_h.at[base_row + RW - 1], sem_out).wait()


_emb = functools.partial(
    pl.kernel,
    out_type=jax.ShapeDtypeStruct((B, S, D), jnp.float32),
    mesh=plsc.VectorSubcoreMesh(core_axis_name="c", subcore_axis_name="s"),
    scratch_types=[
        pltpu.VMEM((VOCAB, D), jnp.float32),
        pltpu.VMEM((MAXPOS, D), jnp.float32),
        pltpu.VMEM((D,), jnp.float32),
        pltpu.VMEM((D,), jnp.float32),
        pltpu.VMEM((2 * VOCAB * MAXPOS + 16,), jnp.float32),
        pltpu.VMEM((RW * S + 64,), jnp.int32),
        pltpu.VMEM((RW, SPAD), jnp.int32),
        pltpu.VMEM((2, SPAD, D), jnp.float32),
        pltpu.SemaphoreType.DMA,
        pltpu.SemaphoreType.DMA,
    ],
)(_body)


def _stats_body(aa_ref, pos_ref, mr_ref):
    aa = aa_ref[...]
    pos = pos_ref[...]
    cd = lax.dot_general(aa, pos, (((1,), (1,)), ((), ())),
                         preferred_element_type=jnp.float32)
    sa = jnp.sum(aa, axis=1, keepdims=True)
    qa = jnp.sum(aa * aa, axis=1, keepdims=True)
    sp = jnp.sum(pos, axis=1)
    qp = jnp.sum(pos * pos, axis=1)
    m = (sa + sp[None, :]) * (1.0 / D)
    q = (qa + qp[None, :] + 2.0 * cd) * (1.0 / D)
    v = jnp.maximum(q - m * m, 0.0) + EPS
    r = lax.rsqrt(v)
    mr_ref[...] = jnp.stack([m, r], axis=-1).reshape(VOCAB, 2 * MAXPOS)


_stats_tc = pl.pallas_call(
    _stats_body,
    out_shape=jax.ShapeDtypeStruct((VOCAB, 2 * MAXPOS), jnp.float32),
)


def kernel(src, AA_emb, Pos_emb, ln_gamma, ln_beta):
    src_flat = src.reshape(-1).astype(jnp.int32)
    mr_tab = _stats_tc(AA_emb, Pos_emb)
    return _emb(src_flat, AA_emb, Pos_emb, ln_gamma, ln_beta,
                mr_tab.reshape(-1))


# speedup vs baseline: 1.4094x; 1.4094x over previous
import functools

import jax
import jax.numpy as jnp
from jax import lax
from jax.experimental import pallas as pl
from jax.experimental.pallas import tpu as pltpu
from jax.experimental.pallas import tpu_sc as plsc

B, S, D = 1024, 200, 128
PAD = 21
VOCAB, MAXPOS = 32, 256
EPS = 1e-12
NC, NS = 2, 16
NW = NC * NS
RW = B // NW
SPAD = 208
NCHUNK = SPAD // 16
NK = D // 16


def _perm(x, idx):
    return x.at[idx].get(mode="promise_in_bounds")


def _splat_sum(x):
    lane = lax.iota(jnp.int32, 16)
    for k in range(4):
        x = x + _perm(x, lane ^ (1 << k))
    return x


def _prefix_sum(x):
    lane = lax.iota(jnp.int32, 16)
    for k in range(4):
        sh = _perm(x, jnp.maximum(lane - (1 << k), 0))
        x = x + jnp.where(lane >= (1 << k), sh, jnp.int32(0))
    return x


def _rsqrt(x):
    i = lax.bitcast_convert_type(x, jnp.int32)
    i = jnp.int32(0x5F3759DF) - lax.shift_right_logical(i, 1)
    y = lax.bitcast_convert_type(i, jnp.float32)
    for _ in range(3):
        y = y * (1.5 - 0.5 * x * y * y)
    return y


def _body(src_h, aa_h, pos_h, ga_h, be_h, out_h,
          aa_v, pos_v, ga_v, be_v, src_v, obuf, sem_in, sem_out):
    wid = lax.axis_index("s") * NC + lax.axis_index("c")
    base_row = wid * RW

    stage = [
        pltpu.make_async_copy(aa_h, aa_v, sem_in),
        pltpu.make_async_copy(pos_h, pos_v, sem_in),
        pltpu.make_async_copy(ga_h, ga_v, sem_in),
        pltpu.make_async_copy(be_h, be_v, sem_in),
        pltpu.make_async_copy(src_h.at[pl.ds(base_row * S, RW * S)],
                              src_v.at[pl.ds(0, RW * S)], sem_in),
    ]
    for c in stage:
        c.start()
    for c in stage:
        c.wait()

    gvec = [ga_v[pl.ds(k * 16, 16)] for k in range(NK)]
    bvec = [be_v[pl.ds(k * 16, 16)] for k in range(NK)]
    last = jnp.full((16,), 15, jnp.int32)

    def row_body(r, _):
        bidx = lax.rem(r, 2)

        @pl.when(r >= 2)
        def _wait_prev():
            pltpu.make_async_copy(obuf.at[bidx, pl.ds(0, S)],
                                  out_h.at[base_row + r - 2], sem_out).wait()

        def chunk_body(i, carry):
            tb = i * 16
            sv_raw = src_v[pl.ds(r * S + tb, 16)]
            sv = jnp.minimum(jnp.maximum(sv_raw, jnp.int32(0)),
                             jnp.int32(VOCAB - 1))
            m = jnp.where(sv_raw == PAD, jnp.int32(0), jnp.int32(1))
            c = _prefix_sum(m) + carry
            pv = jnp.where(m == 1, c, jnp.int32(0))
            pv = jnp.minimum(pv, jnp.int32(MAXPOS - 1))
            for l in range(16):
                s = sv[l]
                p = pv[l]
                e = [aa_v[s, pl.ds(k * 16, 16)] + pos_v[p, pl.ds(k * 16, 16)]
                     for k in range(NK)]
                tot = ((e[0] + e[1]) + (e[2] + e[3])) + \
                      ((e[4] + e[5]) + (e[6] + e[7]))
                sq = [ek * ek for ek in e]
                sqt = ((sq[0] + sq[1]) + (sq[2] + sq[3])) + \
                      ((sq[4] + sq[5]) + (sq[6] + sq[7]))
                mean = _splat_sum(tot) * (1.0 / D)
                ex2 = _splat_sum(sqt) * (1.0 / D)
                var = jnp.maximum(ex2 - mean * mean, 0.0) + EPS
                rstd = _rsqrt(var)
                for k in range(NK):
                    obuf[bidx, tb + l, pl.ds(k * 16, 16)] = \
                        (e[k] - mean) * rstd * gvec[k] + bvec[k]
            return _perm(c, last)

        lax.fori_loop(0, NCHUNK, chunk_body, jnp.zeros((16,), jnp.int32))
        pltpu.make_async_copy(obuf.at[bidx, pl.ds(0, S)],
                              out_h.at[base_row + r], sem_out).start()
        return 0

    lax.fori_loop(0, RW, row_body, 0)
    pltpu.make_async_copy(obuf.at[0, pl.ds(0, S)],
                          out_h.at[base_row + RW - 2], sem_out).wait()
    pltpu.make_async_copy(obuf.at[1, pl.ds(0, S)],
                          out_h.at[base_row + RW - 1], sem_out).wait()


_emb = functools.partial(
    pl.kernel,
    out_type=jax.ShapeDtypeStruct((B, S, D), jnp.float32),
    mesh=plsc.VectorSubcoreMesh(core_axis_name="c", subcore_axis_name="s"),
    scratch_types=[
        pltpu.VMEM((VOCAB, D), jnp.float32),
        pltpu.VMEM((MAXPOS, D), jnp.float32),
        pltpu.VMEM((D,), jnp.float32),
        pltpu.VMEM((D,), jnp.float32),
        pltpu.VMEM((RW * S + 64,), jnp.int32),
        pltpu.VMEM((2, SPAD, D), jnp.float32),
        pltpu.SemaphoreType.DMA,
        pltpu.SemaphoreType.DMA,
    ],
)(_body)


def kernel(src, AA_emb, Pos_emb, ln_gamma, ln_beta):
    src_flat = src.reshape(-1).astype(jnp.int32)
    return _emb(src_flat, AA_emb, Pos_emb, ln_gamma, ln_beta)
